# baseline (speedup 1.0000x reference)
import functools

import jax
import jax.numpy as jnp
from jax import lax
from jax.experimental import pallas as pl
from jax.experimental.pallas import tpu as pltpu
from jax.experimental.pallas import tpu_sc as plsc

NC = 2
NS = 16
NW = NC * NS
L = 16
CB = 1024


def _ceil_to(x, m):
    return (x + m - 1) // m * m


@functools.lru_cache(maxsize=None)
def _make_sc_call(T, N, E):
    assert T == L, "kernel assumes one timestep per vector lane"
    EW = _ceil_to(E, NW * 2 * CB) // NW
    NCHUNK = EW // CB
    NCH2 = NCHUNK // 2
    EP = EW * NW
    NP = _ceil_to(N, NW * 8)
    RW = NP // NW

    mesh = plsc.VectorSubcoreMesh(core_axis_name="c", subcore_axis_name="s")

    def body(tabu, tabv, rows, cols, outpe, outke,
             idxr, idxc, gr0, gc0, vbuf, osc,
             semg0, semg1, semv):
        wid = lax.axis_index("s") * NC + lax.axis_index("c")
        zero = jnp.zeros((L,), jnp.float32)

        def accum(gr, gc, acc):
            def body8(j, accs):
                a0, a1, a2, a3 = accs
                e = j * 8
                d = gr[e] - gc[e]
                a0 = a0 + d * d
                d = gr[e + 1] - gc[e + 1]
                a1 = a1 + d * d
                d = gr[e + 2] - gc[e + 2]
                a2 = a2 + d * d
                d = gr[e + 3] - gc[e + 3]
                a3 = a3 + d * d
                d = gr[e + 4] - gc[e + 4]
                a0 = a0 + d * d
                d = gr[e + 5] - gc[e + 5]
                a1 = a1 + d * d
                d = gr[e + 6] - gc[e + 6]
                a2 = a2 + d * d
                d = gr[e + 7] - gc[e + 7]
                a3 = a3 + d * d
                return (a0, a1, a2, a3)

            accs = lax.fori_loop(0, CB // 8, body8, (zero, zero, zero, zero),
                                 unroll=2)
            return acc + (accs[0] + accs[1]) + (accs[2] + accs[3])

        pltpu.async_copy(tabv.at[pl.ds(wid * RW, RW)], vbuf, semv)

        ebase = wid * NCHUNK * CB

        def chunk(c, acc):
            base = ebase + c * CB
            pltpu.sync_copy(rows.at[pl.ds(base, CB)], idxr)
            pltpu.sync_copy(cols.at[pl.ds(base, CB)], idxc)
            cp1 = pltpu.async_copy(tabu.at[idxr], gr0, semg0)
            cp2 = pltpu.async_copy(tabu.at[idxc], gc0, semg1)
            cp1.wait()
            cp2.wait()
            return accum(gr0, gc0, acc)

        pe = lax.fori_loop(0, NCHUNK, chunk, zero)
        osc[...] = pe
        pltpu.sync_copy(osc, outpe.at[wid])

        pltpu.make_async_copy(tabv.at[pl.ds(wid * RW, RW)], vbuf, semv).wait()

        def krow8(j, accs):
            a0, a1, a2, a3 = accs
            r = j * 8
            x = vbuf[r]
            a0 = a0 + x * x
            x = vbuf[r + 1]
            a1 = a1 + x * x
            x = vbuf[r + 2]
            a2 = a2 + x * x
            x = vbuf[r + 3]
            a3 = a3 + x * x
            x = vbuf[r + 4]
            a0 = a0 + x * x
            x = vbuf[r + 5]
            a1 = a1 + x * x
            x = vbuf[r + 6]
            a2 = a2 + x * x
            x = vbuf[r + 7]
            a3 = a3 + x * x
            return (a0, a1, a2, a3)

        ka = lax.fori_loop(0, RW // 8, krow8, (zero, zero, zero, zero),
                           unroll=2)
        osc[...] = (ka[0] + ka[1]) + (ka[2] + ka[3])
        pltpu.sync_copy(osc, outke.at[wid])

    call = pl.kernel(
        body,
        out_type=(
            jax.ShapeDtypeStruct((NW, L), jnp.float32),
            jax.ShapeDtypeStruct((NW, L), jnp.float32),
        ),
        mesh=mesh,
        scratch_types=[
            pltpu.VMEM((CB,), jnp.int32),
            pltpu.VMEM((CB,), jnp.int32),
            pltpu.VMEM((CB, L), jnp.float32),
            pltpu.VMEM((CB, L), jnp.float32),
            pltpu.VMEM((RW, L), jnp.float32),
            pltpu.VMEM((L,), jnp.float32),
            pltpu.SemaphoreType.DMA,
            pltpu.SemaphoreType.DMA,
            pltpu.SemaphoreType.DMA,
        ],
        compiler_params=pltpu.CompilerParams(use_tc_tiling_on_sc=False),
    )
    return call, EP, NP


def kernel(states, edge_index):
    T, N, _ = states.shape
    E = edge_index.shape[1]
    call, EP, NP = _make_sc_call(T, N, E)

    tabu = states[:, :, 0].T
    tabv = jnp.pad(states[:, :, 1].T, ((0, NP - N), (0, 0)))
    ei = edge_index.astype(jnp.int32)
    eip = jnp.pad(ei, ((0, 0), (0, EP - E)))
    outpe, outke = call(tabu, tabv, eip[0], eip[1])

    H = 0.5 * (jnp.sum(outpe, axis=0) + jnp.sum(outke, axis=0))
    dH = H[1:] - H[:-1]
    return jnp.sum(dH * dH) / (T - 1)


# speedup vs baseline: 1.1321x; 1.1321x over previous
import functools

import jax
import jax.numpy as jnp
from jax import lax
from jax.experimental import pallas as pl
from jax.experimental.pallas import tpu as pltpu
from jax.experimental.pallas import tpu_sc as plsc

NC = 2
NS = 16
NW = NC * NS
L = 16
CB = 1024


def _ceil_to(x, m):
    return (x + m - 1) // m * m


@functools.lru_cache(maxsize=None)
def _make_sc_call(T, N, E):
    assert T == L, "kernel assumes one timestep per vector lane"
    EW = _ceil_to(E, NW * 2 * CB) // NW
    NCHUNK = EW // CB
    NCH2 = NCHUNK // 2
    EP = EW * NW
    NP = _ceil_to(N, NW * 8)
    RW = NP // NW

    mesh = plsc.VectorSubcoreMesh(core_axis_name="c", subcore_axis_name="s")

    def body(tabu, tabv, rows, cols, outpe, outke,
             idxr0, idxc0, idxr1, idxc1, gr0, gc0, gr1, gc1, vbuf, osc,
             semg0, semg1, semv):
        wid = lax.axis_index("s") * NC + lax.axis_index("c")
        zero = jnp.zeros((L,), jnp.float32)

        def accum(gr, gc, acc):
            def edge(e, a):
                du = gr[e] - gc[e]
                return a + du * du

            return acc + lax.fori_loop(0, CB, edge, zero, unroll=8)

        pltpu.async_copy(tabv.at[pl.ds(wid * RW, RW)], vbuf, semv)

        ebase = wid * NCHUNK * CB

        def load_fire(c, idxr, idxc, gr, gc, semg):
            base = ebase + c * CB
            pltpu.sync_copy(rows.at[pl.ds(base, CB)], idxr)
            pltpu.sync_copy(cols.at[pl.ds(base, CB)], idxc)
            pltpu.async_copy(tabu.at[idxr], gr, semg)
            pltpu.async_copy(tabu.at[idxc], gc, semg)

        def g_wait(idxr, idxc, gr, gc, semg):
            pltpu.make_async_copy(tabu.at[idxr], gr, semg).wait()
            pltpu.make_async_copy(tabu.at[idxc], gc, semg).wait()

        load_fire(0, idxr0, idxc0, gr0, gc0, semg0)
        load_fire(1, idxr1, idxc1, gr1, gc1, semg1)

        def chunk2(k, acc):
            g_wait(idxr0, idxc0, gr0, gc0, semg0)
            acc = accum(gr0, gc0, acc)

            @pl.when(k < NCH2 - 1)
            def _():
                load_fire(2 * k + 2, idxr0, idxc0, gr0, gc0, semg0)

            g_wait(idxr1, idxc1, gr1, gc1, semg1)
            acc = accum(gr1, gc1, acc)

            @pl.when(k < NCH2 - 1)
            def _():
                load_fire(2 * k + 3, idxr1, idxc1, gr1, gc1, semg1)

            return acc

        pe = lax.fori_loop(0, NCH2, chunk2, zero)
        osc[...] = pe
        pltpu.sync_copy(osc, outpe.at[wid])

        pltpu.make_async_copy(tabv.at[pl.ds(wid * RW, RW)], vbuf, semv).wait()

        def krow8(j, accs):
            a0, a1, a2, a3 = accs
            r = j * 8
            x = vbuf[r]
            a0 = a0 + x * x
            x = vbuf[r + 1]
            a1 = a1 + x * x
            x = vbuf[r + 2]
            a2 = a2 + x * x
            x = vbuf[r + 3]
            a3 = a3 + x * x
            x = vbuf[r + 4]
            a0 = a0 + x * x
            x = vbuf[r + 5]
            a1 = a1 + x * x
            x = vbuf[r + 6]
            a2 = a2 + x * x
            x = vbuf[r + 7]
            a3 = a3 + x * x
            return (a0, a1, a2, a3)

        ka = lax.fori_loop(0, RW // 8, krow8, (zero, zero, zero, zero),
                           unroll=2)
        osc[...] = (ka[0] + ka[1]) + (ka[2] + ka[3])
        pltpu.sync_copy(osc, outke.at[wid])

    call = pl.kernel(
        body,
        out_type=(
            jax.ShapeDtypeStruct((NW, L), jnp.float32),
            jax.ShapeDtypeStruct((NW, L), jnp.float32),
        ),
        mesh=mesh,
        scratch_types=[
            pltpu.VMEM((CB,), jnp.int32),
            pltpu.VMEM((CB,), jnp.int32),
            pltpu.VMEM((CB,), jnp.int32),
            pltpu.VMEM((CB,), jnp.int32),
            pltpu.VMEM((CB, L), jnp.float32),
            pltpu.VMEM((CB, L), jnp.float32),
            pltpu.VMEM((CB, L), jnp.float32),
            pltpu.VMEM((CB, L), jnp.float32),
            pltpu.VMEM((RW, L), jnp.float32),
            pltpu.VMEM((L,), jnp.float32),
            pltpu.SemaphoreType.DMA,
            pltpu.SemaphoreType.DMA,
            pltpu.SemaphoreType.DMA,
        ],
        compiler_params=pltpu.CompilerParams(use_tc_tiling_on_sc=False),
    )
    return call, EP, NP


def kernel(states, edge_index):
    T, N, _ = states.shape
    E = edge_index.shape[1]
    call, EP, NP = _make_sc_call(T, N, E)

    tabu = states[:, :, 0].T
    tabv = jnp.pad(states[:, :, 1].T, ((0, NP - N), (0, 0)))
    ei = edge_index.astype(jnp.int32)
    eip = jnp.pad(ei, ((0, 0), (0, EP - E)))
    outpe, outke = call(tabu, tabv, eip[0], eip[1])

    H = 0.5 * (jnp.sum(outpe, axis=0) + jnp.sum(outke, axis=0))
    dH = H[1:] - H[:-1]
    return jnp.sum(dH * dH) / (T - 1)
